# baseline (speedup 1.0000x reference)
import functools

import jax
import jax.numpy as jnp
from jax import lax
from jax.experimental import pallas as pl
from jax.experimental.pallas import tpu as pltpu

N = 100000
E = 1600000
H = 64
C = 16
G = 4
BN = 2000
GRID = N // BN


def _rot(x0, x1):
    theta = jnp.float32(jnp.pi / 2)
    c = jnp.cos(theta)
    s = jnp.sin(theta)
    return c * x0 - s * x1, s * x0 + c * x1


def _row16(pairs, fill):
    lane = lax.broadcasted_iota(jnp.int32, (1, 16), 1)
    out = jnp.full((1, 16), fill, jnp.float32)
    for k, v in pairs:
        out = jnp.where(lane == k, v, out)
    return out


def _stats_body(xp_ref, out_ref, acc):
    i = pl.program_id(0)
    xp = xp_ref[...]
    x0 = xp[:, 0:1]
    x1 = xp[:, 1:2]
    ar = xp[:, 2:3]
    rx, ry = _rot(x0, x1)
    sums = _row16([(0, jnp.sum(x0)), (1, jnp.sum(x1)),
                   (2, jnp.sum(rx)), (3, jnp.sum(ry))], 0.0)
    maxs = _row16([(0, jnp.max(x0)), (1, jnp.max(x1)),
                   (2, jnp.max(rx)), (3, jnp.max(ry)),
                   (4, jnp.max(ar))], -jnp.inf)
    mins = _row16([(0, jnp.min(x0)), (1, jnp.min(x1))], jnp.inf)

    @pl.when(i == 0)
    def _():
        acc[0:1, :] = sums
        acc[1:2, :] = maxs
        acc[2:3, :] = mins

    @pl.when(i > 0)
    def _():
        acc[0:1, :] = acc[0:1, :] + sums
        acc[1:2, :] = jnp.maximum(acc[1:2, :], maxs)
        acc[2:3, :] = jnp.minimum(acc[2:3, :], mins)

    @pl.when(i == GRID - 1)
    def _():
        out_ref[...] = acc[...]


def _apply_body(xp_ref, st_ref, out_ref):
    st = st_ref[...]
    xp = xp_ref[...]
    x0 = xp[:, 0:1]
    x1 = xp[:, 1:2]
    ar = xp[:, 2:3]
    rx, ry = _rot(x0, x1)
    rotate = (st[1, 1] - st[2, 1]) > (st[1, 0] - st[2, 0])
    c0 = jnp.where(rotate, rx, x0)
    c1 = jnp.where(rotate, ry, x1)
    mean0 = jnp.where(rotate, st[0, 2], st[0, 0]) / N
    mean1 = jnp.where(rotate, st[0, 3], st[0, 1]) / N
    max0 = jnp.where(rotate, st[1, 2], st[1, 0])
    max1 = jnp.where(rotate, st[1, 3], st[1, 1])
    c0 = (c0 - mean0) / max0
    c1 = (c1 - mean1) / max1
    arn = ar / st[1, 4]
    lane = lax.broadcasted_iota(jnp.int32, xp.shape, 1)
    out = jnp.where(lane == 0, c0, 0.0)
    out = jnp.where(lane == 1, c1, out)
    out = jnp.where(lane == 2, arn, out)
    out_ref[...] = out


def _normalize(xp):
    blk16 = pl.BlockSpec((BN, 16), lambda i: (i, 0))
    st = pl.pallas_call(
        _stats_body,
        grid=(GRID,),
        in_specs=[blk16],
        out_specs=pl.BlockSpec((8, 16), lambda i: (0, 0)),
        out_shape=jax.ShapeDtypeStruct((8, 16), jnp.float32),
        scratch_shapes=[pltpu.VMEM((8, 16), jnp.float32)],
    )(xp)
    return pl.pallas_call(
        _apply_body,
        grid=(GRID,),
        in_specs=[blk16, pl.BlockSpec((8, 16), lambda i: (0, 0))],
        out_specs=blk16,
        out_shape=jax.ShapeDtypeStruct((N, 16), jnp.float32),
    )(xp, st)


def _dense_body(s_ref, cnt_ref, h_ref, wl_ref, bl_ref, wr_ref, out_ref, *tab_refs):
    s = s_ref[...]
    hp = h_ref[...]
    inv = 1.0 / jnp.maximum(cnt_ref[...][:, 0:1], 1.0)
    mean = s * inv
    acc = jnp.dot(mean, wl_ref[...], preferred_element_type=jnp.float32)
    acc += jnp.dot(hp, wr_ref[...], preferred_element_type=jnp.float32)
    h = jnp.tanh(acc + bl_ref[...])
    out_ref[...] = h
    for g, r in enumerate(tab_refs):
        r[...] = h[:, 16 * g:16 * (g + 1)]


def _dense(s_full, cntv, h_full, Wl, bl, Wr):
    blk16 = pl.BlockSpec((BN, 16), lambda i: (i, 0))
    blk64 = pl.BlockSpec((BN, H), lambda i: (i, 0))
    wspec = pl.BlockSpec((H, H), lambda i: (0, 0))
    bspec = pl.BlockSpec((1, H), lambda i: (0, 0))
    return pl.pallas_call(
        _dense_body,
        grid=(GRID,),
        in_specs=[blk64, blk16, blk64, wspec, bspec, wspec],
        out_specs=[blk64] + [blk16] * G,
        out_shape=[jax.ShapeDtypeStruct((N, H), jnp.float32)]
        + [jax.ShapeDtypeStruct((N, 16), jnp.float32)] * G,
    )(s_full, cntv, h_full, Wl.T, bl.reshape(1, H), Wr.T)


def _head_body(*refs):
    (s_ref, cnt_ref, h_ref, wl_ref, bl_ref, wr_ref, w5_ref, b5_ref,
     w6_ref, b6_ref, w7_ref, b7_ref, out_ref) = refs
    s = s_ref[...]
    hp = h_ref[...]
    inv = 1.0 / jnp.maximum(cnt_ref[...][:, 0:1], 1.0)
    acc = jnp.dot(s * inv, wl_ref[...], preferred_element_type=jnp.float32)
    acc += jnp.dot(hp, wr_ref[...], preferred_element_type=jnp.float32)
    h = jnp.tanh(acc + bl_ref[...])
    h = jnp.tanh(jnp.dot(h, w5_ref[...], preferred_element_type=jnp.float32)
                 + b5_ref[...])
    h = jnp.tanh(jnp.dot(h, w6_ref[...], preferred_element_type=jnp.float32)
                 + b6_ref[...])
    logits = jnp.dot(h, w7_ref[...], preferred_element_type=jnp.float32) + b7_ref[...]
    m = jnp.max(logits, axis=1, keepdims=True)
    e = jnp.exp(logits - m)
    out_ref[...] = e / jnp.sum(e, axis=1, keepdims=True)


def _head(s_full, cntv, h_full, Wl, bl, Wr, W5, b5, W6, b6, W7, b7):
    blk16 = pl.BlockSpec((BN, 16), lambda i: (i, 0))
    blk64 = pl.BlockSpec((BN, H), lambda i: (i, 0))
    w64 = pl.BlockSpec((H, H), lambda i: (0, 0))
    b64 = pl.BlockSpec((1, H), lambda i: (0, 0))
    w7s = pl.BlockSpec((H, C), lambda i: (0, 0))
    b7s = pl.BlockSpec((1, C), lambda i: (0, 0))
    return pl.pallas_call(
        _head_body,
        grid=(GRID,),
        in_specs=[blk64, blk16, blk64]
        + [w64, b64, w64, w64, b64, w64, b64, w7s, b7s],
        out_specs=pl.BlockSpec((BN, C), lambda i: (i, 0)),
        out_shape=jax.ShapeDtypeStruct((N, C), jnp.float32),
    )(s_full, cntv, h_full, Wl.T, bl.reshape(1, H), Wr.T,
      W5.T, b5.reshape(1, H), W6.T, b6.reshape(1, H),
      W7.T, b7.reshape(1, C))


def _aggregate(h_full, src, dst):
    msg = jnp.take(h_full, src, axis=0)
    return jax.ops.segment_sum(msg, dst, num_segments=N)


def _degree(dst):
    cnt = jax.ops.segment_sum(jnp.ones((E,), jnp.float32), dst, num_segments=N)
    return jnp.broadcast_to(cnt[:, None], (N, 16))


def kernel(x, edge_index, Wl1, bl1, Wr1, Wl2, bl2, Wr2, Wl3, bl3, Wr3,
           Wl4, bl4, Wr4, W5, b5, W6, b6, W7, b7):
    xp = jnp.pad(x, ((0, 0), (0, 13)))
    h0 = _normalize(xp)
    src = edge_index[0]
    dst = edge_index[1]
    cntv = _degree(dst)
    Wl1p = jnp.pad(Wl1, ((0, 0), (0, 61)))
    Wr1p = jnp.pad(Wr1, ((0, 0), (0, 61)))
    h0f = jnp.pad(h0, ((0, 0), (0, 48)))

    s1 = _aggregate(h0f, src, dst)
    h1, *t1 = _dense(s1, cntv, h0f, Wl1p, bl1, Wr1p)
    s2 = _aggregate(h1, src, dst)
    h2, *t2 = _dense(s2, cntv, h1, Wl2, bl2, Wr2)
    s3 = _aggregate(h2, src, dst)
    h3, *t3 = _dense(s3, cntv, h2, Wl3, bl3, Wr3)
    s4 = _aggregate(h3, src, dst)
    return _head(s4, cntv, h3, Wl4, bl4, Wr4, W5, b5, W6, b6, W7, b7)


# speedup vs baseline: 7.1404x; 7.1404x over previous
import functools

import jax
import jax.numpy as jnp
from jax import lax
from jax.experimental import pallas as pl
from jax.experimental.pallas import tpu as pltpu

N = 100000
E = 1600000
H = 64
C = 16
G = 4
BN = 2000
GRID = N // BN


def _rot(x0, x1):
    theta = jnp.float32(jnp.pi / 2)
    c = jnp.cos(theta)
    s = jnp.sin(theta)
    return c * x0 - s * x1, s * x0 + c * x1


def _row16(pairs, fill):
    lane = lax.broadcasted_iota(jnp.int32, (1, 16), 1)
    out = jnp.full((1, 16), fill, jnp.float32)
    for k, v in pairs:
        out = jnp.where(lane == k, v, out)
    return out


def _stats_body(xp_ref, out_ref, acc):
    i = pl.program_id(0)
    xp = xp_ref[...]
    x0 = xp[:, 0:1]
    x1 = xp[:, 1:2]
    ar = xp[:, 2:3]
    rx, ry = _rot(x0, x1)
    sums = _row16([(0, jnp.sum(x0)), (1, jnp.sum(x1)),
                   (2, jnp.sum(rx)), (3, jnp.sum(ry))], 0.0)
    maxs = _row16([(0, jnp.max(x0)), (1, jnp.max(x1)),
                   (2, jnp.max(rx)), (3, jnp.max(ry)),
                   (4, jnp.max(ar))], -jnp.inf)
    mins = _row16([(0, jnp.min(x0)), (1, jnp.min(x1))], jnp.inf)

    @pl.when(i == 0)
    def _():
        acc[0:1, :] = sums
        acc[1:2, :] = maxs
        acc[2:3, :] = mins

    @pl.when(i > 0)
    def _():
        acc[0:1, :] = acc[0:1, :] + sums
        acc[1:2, :] = jnp.maximum(acc[1:2, :], maxs)
        acc[2:3, :] = jnp.minimum(acc[2:3, :], mins)

    @pl.when(i == GRID - 1)
    def _():
        out_ref[...] = acc[...]


def _apply_body(xp_ref, st_ref, out_ref):
    st = st_ref[...]
    xp = xp_ref[...]
    x0 = xp[:, 0:1]
    x1 = xp[:, 1:2]
    ar = xp[:, 2:3]
    rx, ry = _rot(x0, x1)
    rotate = (st[1, 1] - st[2, 1]) > (st[1, 0] - st[2, 0])
    c0 = jnp.where(rotate, rx, x0)
    c1 = jnp.where(rotate, ry, x1)
    mean0 = jnp.where(rotate, st[0, 2], st[0, 0]) / N
    mean1 = jnp.where(rotate, st[0, 3], st[0, 1]) / N
    max0 = jnp.where(rotate, st[1, 2], st[1, 0])
    max1 = jnp.where(rotate, st[1, 3], st[1, 1])
    c0 = (c0 - mean0) / max0
    c1 = (c1 - mean1) / max1
    arn = ar / st[1, 4]
    lane = lax.broadcasted_iota(jnp.int32, xp.shape, 1)
    out = jnp.where(lane == 0, c0, 0.0)
    out = jnp.where(lane == 1, c1, out)
    out = jnp.where(lane == 2, arn, out)
    out_ref[...] = out


def _normalize(xp):
    blk16 = pl.BlockSpec((BN, 16), lambda i: (i, 0))
    st = pl.pallas_call(
        _stats_body,
        grid=(GRID,),
        in_specs=[blk16],
        out_specs=pl.BlockSpec((8, 16), lambda i: (0, 0)),
        out_shape=jax.ShapeDtypeStruct((8, 16), jnp.float32),
        scratch_shapes=[pltpu.VMEM((8, 16), jnp.float32)],
    )(xp)
    return pl.pallas_call(
        _apply_body,
        grid=(GRID,),
        in_specs=[blk16, pl.BlockSpec((8, 16), lambda i: (0, 0))],
        out_specs=blk16,
        out_shape=jax.ShapeDtypeStruct((N, 16), jnp.float32),
    )(xp, st)


def _dense_body(s_ref, cnt_ref, h_ref, wl_ref, bl_ref, wr_ref, out_ref, *tab_refs):
    s = s_ref[...]
    hp = h_ref[...]
    inv = 1.0 / jnp.maximum(cnt_ref[...][:, 0:1], 1.0)
    mean = s * inv
    acc = jnp.dot(mean, wl_ref[...], preferred_element_type=jnp.float32)
    acc += jnp.dot(hp, wr_ref[...], preferred_element_type=jnp.float32)
    h = jnp.tanh(acc + bl_ref[...])
    out_ref[...] = h
    for g, r in enumerate(tab_refs):
        r[...] = h[:, 16 * g:16 * (g + 1)]


def _dense(s_full, cntv, h_full, Wl, bl, Wr):
    blk16 = pl.BlockSpec((BN, 16), lambda i: (i, 0))
    blk64 = pl.BlockSpec((BN, H), lambda i: (i, 0))
    wspec = pl.BlockSpec((H, H), lambda i: (0, 0))
    bspec = pl.BlockSpec((1, H), lambda i: (0, 0))
    return pl.pallas_call(
        _dense_body,
        grid=(GRID,),
        in_specs=[blk64, blk16, blk64, wspec, bspec, wspec],
        out_specs=[blk64] + [blk16] * G,
        out_shape=[jax.ShapeDtypeStruct((N, H), jnp.float32)]
        + [jax.ShapeDtypeStruct((N, 16), jnp.float32)] * G,
    )(s_full, cntv, h_full, Wl.T, bl.reshape(1, H), Wr.T)


def _head_body(*refs):
    (s_ref, cnt_ref, h_ref, wl_ref, bl_ref, wr_ref, w5_ref, b5_ref,
     w6_ref, b6_ref, w7_ref, b7_ref, out_ref) = refs
    s = s_ref[...]
    hp = h_ref[...]
    inv = 1.0 / jnp.maximum(cnt_ref[...][:, 0:1], 1.0)
    acc = jnp.dot(s * inv, wl_ref[...], preferred_element_type=jnp.float32)
    acc += jnp.dot(hp, wr_ref[...], preferred_element_type=jnp.float32)
    h = jnp.tanh(acc + bl_ref[...])
    h = jnp.tanh(jnp.dot(h, w5_ref[...], preferred_element_type=jnp.float32)
                 + b5_ref[...])
    h = jnp.tanh(jnp.dot(h, w6_ref[...], preferred_element_type=jnp.float32)
                 + b6_ref[...])
    logits = jnp.dot(h, w7_ref[...], preferred_element_type=jnp.float32) + b7_ref[...]
    m = jnp.max(logits, axis=1, keepdims=True)
    e = jnp.exp(logits - m)
    out_ref[...] = e / jnp.sum(e, axis=1, keepdims=True)


def _head(s_full, cntv, h_full, Wl, bl, Wr, W5, b5, W6, b6, W7, b7):
    blk16 = pl.BlockSpec((BN, 16), lambda i: (i, 0))
    blk64 = pl.BlockSpec((BN, H), lambda i: (i, 0))
    w64 = pl.BlockSpec((H, H), lambda i: (0, 0))
    b64 = pl.BlockSpec((1, H), lambda i: (0, 0))
    w7s = pl.BlockSpec((H, C), lambda i: (0, 0))
    b7s = pl.BlockSpec((1, C), lambda i: (0, 0))
    return pl.pallas_call(
        _head_body,
        grid=(GRID,),
        in_specs=[blk64, blk16, blk64]
        + [w64, b64, w64, w64, b64, w64, b64, w7s, b7s],
        out_specs=pl.BlockSpec((BN, C), lambda i: (i, 0)),
        out_shape=jax.ShapeDtypeStruct((N, C), jnp.float32),
    )(s_full, cntv, h_full, Wl.T, bl.reshape(1, H), Wr.T,
      W5.T, b5.reshape(1, H), W6.T, b6.reshape(1, H),
      W7.T, b7.reshape(1, C))


NT = 16
EC = E // NT
K = 800
NCH = EC // K
NPAIR = NCH // 2
Q = 6256
LQ = N - (NT - 1) * Q
ZF = Q // K
ZT = Q - ZF * K
LZT = LQ - ZF * K

_MESH = None


def _mesh():
    global _MESH
    if _MESH is None:
        from jax.experimental.pallas import tpu_sc as plsc
        _MESH = plsc.VectorSubcoreMesh(core_axis_name="c", subcore_axis_name="s",
                                       num_cores=2, num_subcores=NT)
    return _MESH


def _zero_rows(ref, nrows, val=0.0):
    def zb(j, _):
        ref[j, :] = jnp.full((16,), val, jnp.float32)
        return 0
    lax.fori_loop(0, nrows, zb, 0, unroll=False)


def _zero_acc(sid, rows, acc):
    r0 = pl.multiple_of(sid * Q, 8)
    for j in range(ZF):
        pltpu.sync_copy(rows, acc.at[pl.ds(r0 + j * K, K)])

    @pl.when(sid < NT - 1)
    def _():
        pltpu.sync_copy(rows.at[pl.ds(0, ZT)], acc.at[pl.ds(r0 + ZF * K, ZT)])

    @pl.when(sid == NT - 1)
    def _():
        pltpu.sync_copy(rows.at[pl.ds(0, LZT)], acc.at[pl.ds(r0 + ZF * K, LZT)])


def _writeout(sid, acc, dst_ref):
    r0 = pl.multiple_of(sid * Q, 8)

    @pl.when(sid < NT - 1)
    def _():
        pltpu.sync_copy(acc.at[pl.ds(r0, Q)], dst_ref.at[pl.ds(r0, Q)])

    @pl.when(sid == NT - 1)
    def _():
        pltpu.sync_copy(acc.at[pl.ds(r0, LQ)], dst_ref.at[pl.ds(r0, LQ)])


def _agg_pass(tbl, src, dst, sid, srcv, dstv, rows_a, rows_b, acc, sem, *,
              gather=True):
    from jax.experimental.pallas import tpu_sc as plsc
    _zero_rows(rows_a, K)
    _zero_acc(sid, rows_a, acc)
    if not gather:
        _zero_rows(rows_a, K, 1.0)
        _zero_rows(rows_b, K, 1.0)
    plsc.subcore_barrier()

    def one(i, rows):
        base = pl.multiple_of(sid * EC + i * K, 8)
        pltpu.sync_copy(dst.at[pl.ds(base, K)], dstv)
        if gather:
            pltpu.sync_copy(src.at[pl.ds(base, K)], srcv)
            pltpu.async_copy(tbl.at[srcv], rows, sem).wait()
        pltpu.sync_copy(rows, acc.at[dstv], add=True)

    def chunk(p, _):
        one(2 * p, rows_a)
        one(2 * p + 1, rows_b)
        return 0

    lax.fori_loop(0, NPAIR, chunk, 0, unroll=False)
    one(NCH - 1, rows_a)
    plsc.subcore_barrier()


def _agg1_body(h0, src, dst, s_out, cnt_out, srcv, dstv, rows_a, rows_b, acc, sem):
    cid = lax.axis_index("c")
    sid = lax.axis_index("s")

    @pl.when(cid == 0)
    def _():
        _agg_pass(h0, src, dst, sid, srcv, dstv, rows_a, rows_b, acc, sem)
        _writeout(sid, acc, s_out)

    @pl.when(cid == 1)
    def _():
        _agg_pass(h0, src, dst, sid, srcv, dstv, rows_a, rows_b, acc, sem,
                  gather=False)
        _writeout(sid, acc, cnt_out)


def _agg4_body(t0, t1, t2, t3, src, dst, out, srcv, dstv, rows_a, rows_b, acc, sem):
    cid = lax.axis_index("c")
    sid = lax.axis_index("s")
    for half, tabs in enumerate(((t0, t1), (t2, t3))):
        @pl.when(cid == half)
        def _():
            for gi, tbl in enumerate(tabs):
                g = 2 * half + gi
                _agg_pass(tbl, src, dst, sid, srcv, dstv, rows_a, rows_b, acc, sem)
                _writeout(sid, acc, out.at[:, pl.ds(16 * g, 16)])


_SC_SCRATCH = None


def _sc_scratch():
    global _SC_SCRATCH
    if _SC_SCRATCH is None:
        _SC_SCRATCH = [
            pltpu.VMEM((K,), jnp.int32),
            pltpu.VMEM((K,), jnp.int32),
            pltpu.VMEM((K, 16), jnp.float32),
            pltpu.VMEM((K, 16), jnp.float32),
            pltpu.VMEM_SHARED((N, 16), jnp.float32),
            pltpu.SemaphoreType.DMA,
        ]
    return _SC_SCRATCH


def _aggregate1(h0, src, dst):
    f = pl.kernel(
        _agg1_body,
        out_type=(jax.ShapeDtypeStruct((N, 16), jnp.float32),
                  jax.ShapeDtypeStruct((N, 16), jnp.float32)),
        mesh=_mesh(),
        scratch_types=_sc_scratch(),
        compiler_params=pltpu.CompilerParams(use_tc_tiling_on_sc=False),
    )
    return f(h0, src, dst)


def _aggregate4(tabs, src, dst):
    f = pl.kernel(
        _agg4_body,
        out_type=jax.ShapeDtypeStruct((N, H), jnp.float32),
        mesh=_mesh(),
        scratch_types=_sc_scratch(),
        compiler_params=pltpu.CompilerParams(use_tc_tiling_on_sc=False),
    )
    return f(*tabs, src, dst)


def kernel(x, edge_index, Wl1, bl1, Wr1, Wl2, bl2, Wr2, Wl3, bl3, Wr3,
           Wl4, bl4, Wr4, W5, b5, W6, b6, W7, b7):
    xp = jnp.pad(x, ((0, 0), (0, 13)))
    h0 = _normalize(xp)
    src = edge_index[0]
    dst = edge_index[1]
    Wl1p = jnp.pad(Wl1, ((0, 0), (0, 61)))
    Wr1p = jnp.pad(Wr1, ((0, 0), (0, 61)))
    h0f = jnp.pad(h0, ((0, 0), (0, 48)))

    s1_16, cntv = _aggregate1(h0, src, dst)
    s1 = jnp.pad(s1_16, ((0, 0), (0, 48)))
    h1, *t1 = _dense(s1, cntv, h0f, Wl1p, bl1, Wr1p)
    s2 = _aggregate4(t1, src, dst)
    h2, *t2 = _dense(s2, cntv, h1, Wl2, bl2, Wr2)
    s3 = _aggregate4(t2, src, dst)
    h3, *t3 = _dense(s3, cntv, h2, Wl3, bl3, Wr3)
    s4 = _aggregate4(t3, src, dst)
    return _head(s4, cntv, h3, Wl4, bl4, Wr4, W5, b5, W6, b6, W7, b7)


# speedup vs baseline: 8.7900x; 1.2310x over previous
import functools

import jax
import jax.numpy as jnp
from jax import lax
from jax.experimental import pallas as pl
from jax.experimental.pallas import tpu as pltpu

N = 100000
E = 1600000
H = 64
C = 16
G = 4
BN = 2000
GRID = N // BN


def _rot(x0, x1):
    theta = jnp.float32(jnp.pi / 2)
    c = jnp.cos(theta)
    s = jnp.sin(theta)
    return c * x0 - s * x1, s * x0 + c * x1


def _row16(pairs, fill):
    lane = lax.broadcasted_iota(jnp.int32, (1, 16), 1)
    out = jnp.full((1, 16), fill, jnp.float32)
    for k, v in pairs:
        out = jnp.where(lane == k, v, out)
    return out


def _stats_body(xp_ref, out_ref, acc):
    i = pl.program_id(0)
    xp = xp_ref[...]
    x0 = xp[:, 0:1]
    x1 = xp[:, 1:2]
    ar = xp[:, 2:3]
    rx, ry = _rot(x0, x1)
    sums = _row16([(0, jnp.sum(x0)), (1, jnp.sum(x1)),
                   (2, jnp.sum(rx)), (3, jnp.sum(ry))], 0.0)
    maxs = _row16([(0, jnp.max(x0)), (1, jnp.max(x1)),
                   (2, jnp.max(rx)), (3, jnp.max(ry)),
                   (4, jnp.max(ar))], -jnp.inf)
    mins = _row16([(0, jnp.min(x0)), (1, jnp.min(x1))], jnp.inf)

    @pl.when(i == 0)
    def _():
        acc[0:1, :] = sums
        acc[1:2, :] = maxs
        acc[2:3, :] = mins

    @pl.when(i > 0)
    def _():
        acc[0:1, :] = acc[0:1, :] + sums
        acc[1:2, :] = jnp.maximum(acc[1:2, :], maxs)
        acc[2:3, :] = jnp.minimum(acc[2:3, :], mins)

    @pl.when(i == GRID - 1)
    def _():
        out_ref[...] = acc[...]


def _apply_body(xp_ref, st_ref, out_ref):
    st = st_ref[...]
    xp = xp_ref[...]
    x0 = xp[:, 0:1]
    x1 = xp[:, 1:2]
    ar = xp[:, 2:3]
    rx, ry = _rot(x0, x1)
    rotate = (st[1, 1] - st[2, 1]) > (st[1, 0] - st[2, 0])
    c0 = jnp.where(rotate, rx, x0)
    c1 = jnp.where(rotate, ry, x1)
    mean0 = jnp.where(rotate, st[0, 2], st[0, 0]) / N
    mean1 = jnp.where(rotate, st[0, 3], st[0, 1]) / N
    max0 = jnp.where(rotate, st[1, 2], st[1, 0])
    max1 = jnp.where(rotate, st[1, 3], st[1, 1])
    c0 = (c0 - mean0) / max0
    c1 = (c1 - mean1) / max1
    arn = ar / st[1, 4]
    lane = lax.broadcasted_iota(jnp.int32, xp.shape, 1)
    out = jnp.where(lane == 0, c0, 0.0)
    out = jnp.where(lane == 1, c1, out)
    out = jnp.where(lane == 2, arn, out)
    out_ref[...] = out


def _normalize(xp):
    blk16 = pl.BlockSpec((BN, 16), lambda i: (i, 0))
    st = pl.pallas_call(
        _stats_body,
        grid=(GRID,),
        in_specs=[blk16],
        out_specs=pl.BlockSpec((8, 16), lambda i: (0, 0)),
        out_shape=jax.ShapeDtypeStruct((8, 16), jnp.float32),
        scratch_shapes=[pltpu.VMEM((8, 16), jnp.float32)],
    )(xp)
    return pl.pallas_call(
        _apply_body,
        grid=(GRID,),
        in_specs=[blk16, pl.BlockSpec((8, 16), lambda i: (0, 0))],
        out_specs=blk16,
        out_shape=jax.ShapeDtypeStruct((N, 16), jnp.float32),
    )(xp, st)


def _dense_body(s_ref, cnt_ref, h_ref, wl_ref, bl_ref, wr_ref, out_ref, *tab_refs):
    s = s_ref[...]
    hp = h_ref[...]
    inv = 1.0 / jnp.maximum(cnt_ref[...][:, 0:1], 1.0)
    mean = s * inv
    acc = jnp.dot(mean, wl_ref[...], preferred_element_type=jnp.float32)
    acc += jnp.dot(hp, wr_ref[...], preferred_element_type=jnp.float32)
    h = jnp.tanh(acc + bl_ref[...])
    out_ref[...] = h
    for g, r in enumerate(tab_refs):
        r[...] = h[:, 16 * g:16 * (g + 1)]


def _dense(s_full, cntv, h_full, Wl, bl, Wr):
    blk16 = pl.BlockSpec((BN, 16), lambda i: (i, 0))
    blk64 = pl.BlockSpec((BN, H), lambda i: (i, 0))
    wspec = pl.BlockSpec((H, H), lambda i: (0, 0))
    bspec = pl.BlockSpec((1, H), lambda i: (0, 0))
    return pl.pallas_call(
        _dense_body,
        grid=(GRID,),
        in_specs=[blk64, blk16, blk64, wspec, bspec, wspec],
        out_specs=[blk64] + [blk16] * G,
        out_shape=[jax.ShapeDtypeStruct((N, H), jnp.float32)]
        + [jax.ShapeDtypeStruct((N, 16), jnp.float32)] * G,
    )(s_full, cntv, h_full, Wl.T, bl.reshape(1, H), Wr.T)


def _head_body(*refs):
    (s_ref, cnt_ref, h_ref, wl_ref, bl_ref, wr_ref, w5_ref, b5_ref,
     w6_ref, b6_ref, w7_ref, b7_ref, out_ref) = refs
    s = s_ref[...]
    hp = h_ref[...]
    inv = 1.0 / jnp.maximum(cnt_ref[...][:, 0:1], 1.0)
    acc = jnp.dot(s * inv, wl_ref[...], preferred_element_type=jnp.float32)
    acc += jnp.dot(hp, wr_ref[...], preferred_element_type=jnp.float32)
    h = jnp.tanh(acc + bl_ref[...])
    h = jnp.tanh(jnp.dot(h, w5_ref[...], preferred_element_type=jnp.float32)
                 + b5_ref[...])
    h = jnp.tanh(jnp.dot(h, w6_ref[...], preferred_element_type=jnp.float32)
                 + b6_ref[...])
    logits = jnp.dot(h, w7_ref[...], preferred_element_type=jnp.float32) + b7_ref[...]
    m = jnp.max(logits, axis=1, keepdims=True)
    e = jnp.exp(logits - m)
    out_ref[...] = e / jnp.sum(e, axis=1, keepdims=True)


def _head(s_full, cntv, h_full, Wl, bl, Wr, W5, b5, W6, b6, W7, b7):
    blk16 = pl.BlockSpec((BN, 16), lambda i: (i, 0))
    blk64 = pl.BlockSpec((BN, H), lambda i: (i, 0))
    w64 = pl.BlockSpec((H, H), lambda i: (0, 0))
    b64 = pl.BlockSpec((1, H), lambda i: (0, 0))
    w7s = pl.BlockSpec((H, C), lambda i: (0, 0))
    b7s = pl.BlockSpec((1, C), lambda i: (0, 0))
    return pl.pallas_call(
        _head_body,
        grid=(GRID,),
        in_specs=[blk64, blk16, blk64]
        + [w64, b64, w64, w64, b64, w64, b64, w7s, b7s],
        out_specs=pl.BlockSpec((BN, C), lambda i: (i, 0)),
        out_shape=jax.ShapeDtypeStruct((N, C), jnp.float32),
    )(s_full, cntv, h_full, Wl.T, bl.reshape(1, H), Wr.T,
      W5.T, b5.reshape(1, H), W6.T, b6.reshape(1, H),
      W7.T, b7.reshape(1, C))


NT = 16
EC = E // NT
K = 800
NCH = EC // K
NPAIR = NCH // 2
Q = 6256
LQ = N - (NT - 1) * Q
ZF = Q // K
ZT = Q - ZF * K
LZT = LQ - ZF * K

_MESH = None


def _mesh():
    global _MESH
    if _MESH is None:
        from jax.experimental.pallas import tpu_sc as plsc
        _MESH = plsc.VectorSubcoreMesh(core_axis_name="c", subcore_axis_name="s",
                                       num_cores=2, num_subcores=NT)
    return _MESH


def _zero_rows(ref, nrows, val=0.0):
    def zb(j, _):
        ref[j, :] = jnp.full((16,), val, jnp.float32)
        return 0
    lax.fori_loop(0, nrows, zb, 0, unroll=False)


def _zero_acc(sid, rows, acc):
    r0 = pl.multiple_of(sid * Q, 8)
    for j in range(ZF):
        pltpu.sync_copy(rows, acc.at[pl.ds(r0 + j * K, K)])

    @pl.when(sid < NT - 1)
    def _():
        pltpu.sync_copy(rows.at[pl.ds(0, ZT)], acc.at[pl.ds(r0 + ZF * K, ZT)])

    @pl.when(sid == NT - 1)
    def _():
        pltpu.sync_copy(rows.at[pl.ds(0, LZT)], acc.at[pl.ds(r0 + ZF * K, LZT)])


def _writeout(sid, acc, dst_ref):
    r0 = pl.multiple_of(sid * Q, 8)

    @pl.when(sid < NT - 1)
    def _():
        pltpu.sync_copy(acc.at[pl.ds(r0, Q)], dst_ref.at[pl.ds(r0, Q)])

    @pl.when(sid == NT - 1)
    def _():
        pltpu.sync_copy(acc.at[pl.ds(r0, LQ)], dst_ref.at[pl.ds(r0, LQ)])


def _agg_pass(tbl, src, dst, sid, bufs_a, bufs_b, acc, *, gather=True):
    from jax.experimental.pallas import tpu_sc as plsc
    srcv_a, dstv_a, rows_a, sem_a = bufs_a
    srcv_b, dstv_b, rows_b, sem_b = bufs_b
    _zero_rows(rows_a, K)
    _zero_acc(sid, rows_a, acc)
    if not gather:
        _zero_rows(rows_a, K, 1.0)
        _zero_rows(rows_b, K, 1.0)
    plsc.subcore_barrier()

    def start(i, srcv, dstv, rows, sem):
        base = pl.multiple_of(sid * EC + i * K, 8)
        pltpu.sync_copy(dst.at[pl.ds(base, K)], dstv)
        if gather:
            pltpu.sync_copy(src.at[pl.ds(base, K)], srcv)
            return pltpu.async_copy(tbl.at[srcv], rows, sem)
        return None

    def chunk(p, _):
        ha = start(2 * p, srcv_a, dstv_a, rows_a, sem_a)
        hb = start(2 * p + 1, srcv_b, dstv_b, rows_b, sem_b)
        if ha is not None:
            ha.wait()
        pltpu.sync_copy(rows_a, acc.at[dstv_a], add=True)
        if hb is not None:
            hb.wait()
        pltpu.sync_copy(rows_b, acc.at[dstv_b], add=True)
        return 0

    lax.fori_loop(0, NPAIR, chunk, 0, unroll=False)
    ht = start(NCH - 1, srcv_a, dstv_a, rows_a, sem_a)
    if ht is not None:
        ht.wait()
    pltpu.sync_copy(rows_a, acc.at[dstv_a], add=True)
    plsc.subcore_barrier()


def _agg1_body(h0, src, dst, s_out, cnt_out, srcv_a, dstv_a, rows_a, srcv_b,
               dstv_b, rows_b, acc, sem_a, sem_b):
    cid = lax.axis_index("c")
    sid = lax.axis_index("s")
    bufs_a = (srcv_a, dstv_a, rows_a, sem_a)
    bufs_b = (srcv_b, dstv_b, rows_b, sem_b)

    @pl.when(cid == 0)
    def _():
        _agg_pass(h0, src, dst, sid, bufs_a, bufs_b, acc)
        _writeout(sid, acc, s_out)

    @pl.when(cid == 1)
    def _():
        _agg_pass(h0, src, dst, sid, bufs_a, bufs_b, acc, gather=False)
        _writeout(sid, acc, cnt_out)


def _agg4_body(t0, t1, t2, t3, src, dst, out, srcv_a, dstv_a, rows_a, srcv_b,
               dstv_b, rows_b, acc, sem_a, sem_b):
    cid = lax.axis_index("c")
    sid = lax.axis_index("s")
    bufs_a = (srcv_a, dstv_a, rows_a, sem_a)
    bufs_b = (srcv_b, dstv_b, rows_b, sem_b)
    for half, tabs in enumerate(((t0, t1), (t2, t3))):
        @pl.when(cid == half)
        def _():
            for gi, tbl in enumerate(tabs):
                g = 2 * half + gi
                _agg_pass(tbl, src, dst, sid, bufs_a, bufs_b, acc)
                _writeout(sid, acc, out.at[:, pl.ds(16 * g, 16)])


_SC_SCRATCH = None


def _sc_scratch():
    global _SC_SCRATCH
    if _SC_SCRATCH is None:
        _SC_SCRATCH = [
            pltpu.VMEM((K,), jnp.int32),
            pltpu.VMEM((K,), jnp.int32),
            pltpu.VMEM((K, 16), jnp.float32),
            pltpu.VMEM((K,), jnp.int32),
            pltpu.VMEM((K,), jnp.int32),
            pltpu.VMEM((K, 16), jnp.float32),
            pltpu.VMEM_SHARED((N, 16), jnp.float32),
            pltpu.SemaphoreType.DMA,
            pltpu.SemaphoreType.DMA,
        ]
    return _SC_SCRATCH


def _aggregate1(h0, src, dst):
    f = pl.kernel(
        _agg1_body,
        out_type=(jax.ShapeDtypeStruct((N, 16), jnp.float32),
                  jax.ShapeDtypeStruct((N, 16), jnp.float32)),
        mesh=_mesh(),
        scratch_types=_sc_scratch(),
        compiler_params=pltpu.CompilerParams(use_tc_tiling_on_sc=False),
    )
    return f(h0, src, dst)


def _aggregate4(tabs, src, dst):
    f = pl.kernel(
        _agg4_body,
        out_type=jax.ShapeDtypeStruct((N, H), jnp.float32),
        mesh=_mesh(),
        scratch_types=_sc_scratch(),
        compiler_params=pltpu.CompilerParams(use_tc_tiling_on_sc=False),
    )
    return f(*tabs, src, dst)


def kernel(x, edge_index, Wl1, bl1, Wr1, Wl2, bl2, Wr2, Wl3, bl3, Wr3,
           Wl4, bl4, Wr4, W5, b5, W6, b6, W7, b7):
    xp = jnp.pad(x, ((0, 0), (0, 13)))
    h0 = _normalize(xp)
    src = edge_index[0]
    dst = edge_index[1]
    Wl1p = jnp.pad(Wl1, ((0, 0), (0, 61)))
    Wr1p = jnp.pad(Wr1, ((0, 0), (0, 61)))
    h0f = jnp.pad(h0, ((0, 0), (0, 48)))

    s1_16, cntv = _aggregate1(h0, src, dst)
    s1 = jnp.pad(s1_16, ((0, 0), (0, 48)))
    h1, *t1 = _dense(s1, cntv, h0f, Wl1p, bl1, Wr1p)
    s2 = _aggregate4(t1, src, dst)
    h2, *t2 = _dense(s2, cntv, h1, Wl2, bl2, Wr2)
    s3 = _aggregate4(t2, src, dst)
    h3, *t3 = _dense(s3, cntv, h2, Wl3, bl3, Wr3)
    s4 = _aggregate4(t3, src, dst)
    return _head(s4, cntv, h3, Wl4, bl4, Wr4, W5, b5, W6, b6, W7, b7)
